# baseline (speedup 1.0000x reference)
import functools

import jax
import jax.numpy as jnp
from jax import lax
from jax.experimental import pallas as pl
from jax.experimental.pallas import tpu as pltpu
from jax.experimental.pallas import tpu_sc as plsc

POOL_SIZE = 1000000
LATENT_DIM = 64
BATCH = 16384

_info = plsc.get_sparse_core_info()
_NC, _NS = _info.num_cores, _info.num_subcores
_NW = _NC * _NS
_BPW = BATCH // _NW

_mesh = plsc.VectorSubcoreMesh(core_axis_name="c", subcore_axis_name="s")


@functools.partial(
    pl.kernel,
    mesh=_mesh,
    out_type=jax.ShapeDtypeStruct((BATCH, LATENT_DIM), jnp.float32),
    scratch_types=[
        pltpu.VMEM((_BPW,), jnp.int32),
        pltpu.VMEM((_BPW, LATENT_DIM), jnp.float32),
        pltpu.SemaphoreType.DMA,
    ],
    compiler_params=pltpu.CompilerParams(use_tc_tiling_on_sc=False),
)
def _gather_sc(idx_hbm, table_hbm, out_hbm, idx_v, rows_v, sem):
    wid = lax.axis_index("s") * _NC + lax.axis_index("c")
    base = wid * _BPW
    pltpu.sync_copy(idx_hbm.at[pl.ds(base, _BPW)], idx_v)
    pltpu.async_copy(table_hbm.at[idx_v], rows_v, sem).wait()
    pltpu.sync_copy(rows_v, out_hbm.at[pl.ds(base, _BPW)])


def kernel(indices, latents):
    return _gather_sc(indices.astype(jnp.int32), latents)


# speedup vs baseline: 1.1365x; 1.1365x over previous
import functools

import jax
import jax.numpy as jnp
from jax import lax
from jax.experimental import pallas as pl
from jax.experimental.pallas import tpu as pltpu
from jax.experimental.pallas import tpu_sc as plsc

POOL_SIZE = 1000000
LATENT_DIM = 64
BATCH = 16384

_info = plsc.get_sparse_core_info()
_NC, _NS, _L = _info.num_cores, _info.num_subcores, _info.num_lanes
_NW = _NC * _NS
_BPW = BATCH // _NW

_mesh = plsc.VectorSubcoreMesh(core_axis_name="c", subcore_axis_name="s")


@functools.partial(
    pl.kernel,
    mesh=_mesh,
    out_type=jax.ShapeDtypeStruct((BATCH // 8, 8, LATENT_DIM), jnp.float32),
    scratch_types=[
        pltpu.VMEM((_BPW,), jnp.int32),
        pltpu.VMEM((8, LATENT_DIM), jnp.float32),
        pltpu.VMEM((2, 8, LATENT_DIM), jnp.float32),
    ],
    compiler_params=pltpu.CompilerParams(needs_layout_passes=False),
)
def _gather_sc(table_hbm, idx_hbm, out_hbm, idx_v, tile_v, sel_v):
    wid = lax.axis_index("s") * _NC + lax.axis_index("c")
    base = wid * _BPW
    pltpu.sync_copy(idx_hbm.at[pl.ds(base, _BPW)], idx_v)

    lane = lax.iota(jnp.int32, _L)

    def row_body(j, _):
        grp = j // _L
        slot = j % _L
        idx16 = idx_v[pl.ds(grp * _L, _L)]
        sel = jnp.where(lane == slot, idx16, 0)
        i_scalar = jnp.sum(sel)
        t = i_scalar >> 3
        r = i_scalar & 7
        pltpu.sync_copy(table_hbm.at[t], tile_v)
        for c in range(LATENT_DIM // _L):
            sel_v[(j % 16) // 8, j % 8, pl.ds(c * _L, _L)] = (
                tile_v[r, pl.ds(c * _L, _L)])
        @pl.when(slot == _L - 1)
        def _():
            pltpu.sync_copy(
                sel_v, out_hbm.at[pl.ds(wid * (_BPW // 8) + grp * 2, 2)])
        return 0

    lax.fori_loop(0, _BPW, row_body, 0)


def kernel(indices, latents):
    out3 = _gather_sc(latents.reshape(POOL_SIZE // 8, 8, LATENT_DIM),
                      indices.astype(jnp.int32))
    return out3.reshape(BATCH, LATENT_DIM)


# speedup vs baseline: 2.2064x; 1.9414x over previous
import functools

import jax
import jax.numpy as jnp
from jax import lax
from jax.experimental import pallas as pl
from jax.experimental.pallas import tpu as pltpu
from jax.experimental.pallas import tpu_sc as plsc

POOL_SIZE = 1000000
LATENT_DIM = 64
BATCH = 16384

_info = plsc.get_sparse_core_info()
_NC, _NS, _L = _info.num_cores, _info.num_subcores, _info.num_lanes
_NW = _NC * _NS
_BPW = BATCH // _NW
_G = _L
_NG = _BPW // _G

_mesh = plsc.VectorSubcoreMesh(core_axis_name="c", subcore_axis_name="s")


@functools.partial(
    pl.kernel,
    mesh=_mesh,
    out_type=jax.ShapeDtypeStruct((BATCH // 8, 8, LATENT_DIM), jnp.float32),
    scratch_types=[
        pltpu.VMEM((_BPW,), jnp.int32),
        pltpu.VMEM((_G, 8, LATENT_DIM), jnp.float32),
        pltpu.VMEM((_G, 8, LATENT_DIM), jnp.float32),
        pltpu.VMEM((_G // 8, 8, LATENT_DIM), jnp.float32),
        pltpu.VMEM((_G // 8, 8, LATENT_DIM), jnp.float32),
        pltpu.SemaphoreType.DMA,
        pltpu.SemaphoreType.DMA,
        pltpu.SemaphoreType.DMA,
        pltpu.SemaphoreType.DMA,
    ],
    compiler_params=pltpu.CompilerParams(needs_layout_passes=False),
)
def _gather_sc(table_hbm, idx_hbm, out_hbm, idx_v, tiles0_v, tiles1_v,
               sel0_v, sel1_v, gsem0, gsem1, wsem0, wsem1):
    wid = lax.axis_index("s") * _NC + lax.axis_index("c")
    base = wid * _BPW
    pltpu.sync_copy(idx_hbm.at[pl.ds(base, _BPW)], idx_v)

    lane = lax.iota(jnp.int32, _L)
    d0 = lane >> 3
    d1 = lane & 7
    tile_rings = (tiles0_v, tiles1_v)
    sel_rings = (sel0_v, sel1_v)
    gsems = (gsem0, gsem1)
    wsems = (wsem0, wsem1)

    def fire(g, ring):
        idx16 = idx_v[pl.ds(g * _G, _G)]
        for s in range(_G):
            i = jnp.sum(jnp.where(lane == s, idx16, 0))
            pltpu.async_copy(table_hbm.at[i >> 3], tile_rings[ring].at[s],
                             gsems[ring])

    def process(g, h, ring):
        for s in range(_G):
            pltpu.make_async_copy(table_hbm.at[0], tile_rings[ring].at[s],
                                  gsems[ring]).wait()
        @pl.when(h >= 1)
        def _():
            pltpu.make_async_copy(
                sel_rings[ring], out_hbm.at[pl.ds(0, _G // 8)],
                wsems[ring]).wait()
        sub = idx_v[pl.ds(g * _G, _G)] & 7

        def col_body(c, carry):
            cvec = jnp.full((_L,), c, jnp.int32)
            vals = plsc.load_gather(tile_rings[ring], [lane, sub, cvec])
            plsc.store_scatter(sel_rings[ring], [d0, d1, cvec], vals)
            return carry

        lax.fori_loop(0, LATENT_DIM, col_body, 0, unroll=8)
        pltpu.async_copy(
            sel_rings[ring],
            out_hbm.at[pl.ds(wid * (_BPW // 8) + g * (_G // 8), _G // 8)],
            wsems[ring])

    fire(0, 0)

    def body2(h, carry):
        g0 = 2 * h
        fire(g0 + 1, 1)
        process(g0, h, 0)

        @pl.when(h < _NG // 2 - 1)
        def _():
            fire(g0 + 2, 0)

        process(g0 + 1, h, 1)
        return carry

    lax.fori_loop(0, _NG // 2, body2, 0)

    for ring in range(2):
        pltpu.make_async_copy(sel_rings[ring], out_hbm.at[pl.ds(0, _G // 8)],
                              wsems[ring]).wait()


def kernel(indices, latents):
    out3 = _gather_sc(latents.reshape(POOL_SIZE // 8, 8, LATENT_DIM),
                      indices.astype(jnp.int32))
    return out3.reshape(BATCH, LATENT_DIM)
